# baseline (speedup 1.0000x reference)
import functools

import jax
import jax.numpy as jnp
from jax import lax
from jax.experimental import pallas as pl
from jax.experimental.pallas import tpu as pltpu
from jax.experimental.pallas import tpu_sc as plsc

N_NODES = 10000
NFEAT = 128
N_EDGES = 320000
NCLASS = 40

NC = 2
NS = 16
NW = NC * NS
EDGES_PER_TILE = N_EDGES // NW
CHUNK = 100
NCHUNK = EDGES_PER_TILE // CHUNK
IBLK = 10
NBLK = NCHUNK // IBLK
NBUF = 3
ROWS_PER_TILE = 624
ROWS_TAIL = N_NODES - NS * ROWS_PER_TILE

_sc_mesh = plsc.VectorSubcoreMesh(core_axis_name="c", subcore_axis_name="s")


@functools.partial(
    pl.kernel,
    mesh=_sc_mesh,
    out_type=jax.ShapeDtypeStruct((NC * N_NODES, NFEAT), jnp.float32),
    scratch_types=[
        pltpu.VMEM((IBLK, CHUNK), jnp.int32),
        pltpu.VMEM((IBLK, CHUNK), jnp.int32),
        pltpu.VMEM((CHUNK, NFEAT), jnp.float32),
        pltpu.VMEM((CHUNK, NFEAT), jnp.float32),
        pltpu.VMEM((CHUNK, NFEAT), jnp.float32),
        pltpu.VMEM_SHARED((N_NODES, NFEAT), jnp.float32),
        pltpu.SemaphoreType.DMA,
        pltpu.SemaphoreType.DMA,
        pltpu.SemaphoreType.DMA,
        pltpu.SemaphoreType.DMA,
        pltpu.SemaphoreType.DMA,
        pltpu.SemaphoreType.DMA,
    ],
)
def _sc_aggregate(x_hbm, src_hbm, dst_hbm, zero_hbm, out_hbm,
                  src_v, dst_v, rows0, rows1, rows2, acc_sh,
                  gs0, gs1, gs2, ss0, ss1, ss2):
    cid = lax.axis_index("c")
    sid = lax.axis_index("s")
    wid = sid * NC + cid

    @pl.when(jnp.logical_and(sid == 0, cid == 0))
    def _():
        pltpu.sync_copy(x_hbm, acc_sh)

    @pl.when(jnp.logical_and(sid == 0, cid == 1))
    def _():
        pltpu.sync_copy(zero_hbm, acc_sh)

    plsc.subcore_barrier()

    rows = (rows0, rows1, rows2)
    gsem = (gs0, gs1, gs2)
    ssem = (ss0, ss1, ss2)

    def blk_body(b, carry):
        pltpu.sync_copy(src_hbm.at[wid, b], src_v)
        pltpu.sync_copy(dst_hbm.at[wid, b], dst_v)

        for k in range(NBUF):
            pltpu.async_copy(x_hbm.at[src_v.at[k]], rows[k], gsem[k])

        for k in range(IBLK):
            m = k % NBUF
            if k >= 1 and k + NBUF - 1 < IBLK:
                p = (k - 1) % NBUF
                pltpu.make_async_copy(rows[p], acc_sh.at[dst_v.at[k - 1]],
                                      ssem[p]).wait()
                pltpu.async_copy(x_hbm.at[src_v.at[k + NBUF - 1]],
                                 rows[p], gsem[p])
            pltpu.make_async_copy(x_hbm.at[src_v.at[k]], rows[m],
                                  gsem[m]).wait()
            pltpu.async_copy(rows[m], acc_sh.at[dst_v.at[k]], ssem[m],
                             add=True)

        for k in range(IBLK - NBUF, IBLK):
            m = k % NBUF
            pltpu.make_async_copy(rows[m], acc_sh.at[dst_v.at[k]],
                                  ssem[m]).wait()
        return carry

    lax.fori_loop(0, NBLK, blk_body, 0)

    plsc.subcore_barrier()

    pltpu.sync_copy(
        acc_sh.at[pl.ds(sid * ROWS_PER_TILE, ROWS_PER_TILE)],
        out_hbm.at[pl.ds(cid * N_NODES + sid * ROWS_PER_TILE, ROWS_PER_TILE)],
    )

    @pl.when(sid == 0)
    def _():
        pltpu.sync_copy(
            acc_sh.at[pl.ds(NS * ROWS_PER_TILE, ROWS_TAIL)],
            out_hbm.at[pl.ds(cid * N_NODES + NS * ROWS_PER_TILE, ROWS_TAIL)],
        )


def _matmul_t(h, W):
    return lax.dot_general(h, W, (((1,), (1,)), ((), ())),
                           preferred_element_type=jnp.float32)


def _tc_mlp(agg_ref, W1_ref, b1_ref, g_ref, be_ref,
            W2_ref, b2_ref, Wfc_ref, bfc_ref, o_ref):
    h = agg_ref[0:N_NODES, :] + agg_ref[N_NODES:, :]
    h = jnp.maximum(_matmul_t(h, W1_ref[...]) + b1_ref[...], 0.0)
    mean = jnp.mean(h, axis=0, keepdims=True)
    var = jnp.mean((h - mean) ** 2, axis=0, keepdims=True)
    h = (h - mean) * (g_ref[...] * lax.rsqrt(var + 1e-5)) + be_ref[...]
    h = _matmul_t(h, W2_ref[...]) + b2_ref[...]
    o_ref[...] = _matmul_t(h, Wfc_ref[...]) + bfc_ref[...]


def kernel(x, edge_index, W1, b1, gamma, beta, W2, b2, Wfc, bfc):
    x = x.astype(jnp.float32)
    ei = edge_index.astype(jnp.int32)
    src = ei[0].reshape(NW, NBLK, IBLK, CHUNK)
    dst = ei[1].reshape(NW, NBLK, IBLK, CHUNK)
    zero = jnp.zeros((N_NODES, NFEAT), jnp.float32)

    agg = _sc_aggregate(x, src, dst, zero)

    out = pl.pallas_call(
        _tc_mlp,
        out_shape=jax.ShapeDtypeStruct((N_NODES, NCLASS), jnp.float32),
    )(agg, W1, b1.reshape(1, -1), gamma.reshape(1, -1),
      beta.reshape(1, -1), W2, b2.reshape(1, -1), Wfc, bfc.reshape(1, -1))
    return out


# speedup vs baseline: 1.0683x; 1.0683x over previous
import functools

import jax
import jax.numpy as jnp
from jax import lax
from jax.experimental import pallas as pl
from jax.experimental.pallas import tpu as pltpu
from jax.experimental.pallas import tpu_sc as plsc

N_NODES = 10000
NFEAT = 128
N_EDGES = 320000
NCLASS = 40

NC = 2
NS = 16
NW = NC * NS
EDGES_PER_TILE = N_EDGES // NW
CHUNK = 125
NCHUNK = EDGES_PER_TILE // CHUNK
IBLK = 40
NBLK = NCHUNK // IBLK
ROWS_PER_TILE = 624
ROWS_TAIL = N_NODES - NS * ROWS_PER_TILE

_sc_mesh = plsc.VectorSubcoreMesh(core_axis_name="c", subcore_axis_name="s")


@functools.partial(
    pl.kernel,
    mesh=_sc_mesh,
    out_type=jax.ShapeDtypeStruct((NC * N_NODES, NFEAT), jnp.float32),
    scratch_types=[
        pltpu.VMEM((IBLK, CHUNK), jnp.int32),
        pltpu.VMEM((IBLK, CHUNK), jnp.int32),
        pltpu.VMEM((CHUNK, NFEAT), jnp.float32),
        pltpu.VMEM((CHUNK, NFEAT), jnp.float32),
        pltpu.VMEM_SHARED((N_NODES, NFEAT), jnp.float32),
        pltpu.SemaphoreType.DMA,
        pltpu.SemaphoreType.DMA,
    ],
)
def _sc_aggregate(x_hbm, src_hbm, dst_hbm, zero_hbm, out_hbm,
                  src_v, dst_v, rows_a, rows_b, acc_sh, sem_a, sem_b):
    cid = lax.axis_index("c")
    sid = lax.axis_index("s")
    wid = sid * NC + cid

    @pl.when(jnp.logical_and(sid == 0, cid == 0))
    def _():
        pltpu.sync_copy(x_hbm, acc_sh)

    @pl.when(jnp.logical_and(sid == 0, cid == 1))
    def _():
        pltpu.sync_copy(zero_hbm, acc_sh)

    plsc.subcore_barrier()

    def blk_body(b, carry):
        pltpu.sync_copy(src_hbm.at[wid, b], src_v)
        pltpu.sync_copy(dst_hbm.at[wid, b], dst_v)
        pltpu.async_copy(x_hbm.at[src_v.at[0]], rows_a, sem_a)

        def body(i, carry2):
            j = 2 * i
            pltpu.async_copy(x_hbm.at[src_v.at[j + 1]], rows_b, sem_b)
            pltpu.make_async_copy(x_hbm.at[src_v.at[j]], rows_a,
                                  sem_a).wait()
            pltpu.sync_copy(rows_a, acc_sh.at[dst_v.at[j]], add=True)

            @pl.when(j + 2 < IBLK)
            def _():
                pltpu.async_copy(x_hbm.at[src_v.at[j + 2]], rows_a, sem_a)

            pltpu.make_async_copy(x_hbm.at[src_v.at[j + 1]], rows_b,
                                  sem_b).wait()
            pltpu.sync_copy(rows_b, acc_sh.at[dst_v.at[j + 1]], add=True)
            return carry2

        lax.fori_loop(0, IBLK // 2, body, 0)
        return carry

    lax.fori_loop(0, NBLK, blk_body, 0)

    plsc.subcore_barrier()

    pltpu.sync_copy(
        acc_sh.at[pl.ds(sid * ROWS_PER_TILE, ROWS_PER_TILE)],
        out_hbm.at[pl.ds(cid * N_NODES + sid * ROWS_PER_TILE, ROWS_PER_TILE)],
    )

    @pl.when(sid == 0)
    def _():
        pltpu.sync_copy(
            acc_sh.at[pl.ds(NS * ROWS_PER_TILE, ROWS_TAIL)],
            out_hbm.at[pl.ds(cid * N_NODES + NS * ROWS_PER_TILE, ROWS_TAIL)],
        )


def _matmul_t(h, W):
    return lax.dot_general(h, W, (((1,), (1,)), ((), ())),
                           preferred_element_type=jnp.float32)


def _tc_mlp(agg_ref, W1_ref, b1_ref, g_ref, be_ref,
            W2_ref, b2_ref, Wfc_ref, bfc_ref, o_ref):
    h = agg_ref[0:N_NODES, :] + agg_ref[N_NODES:, :]
    h = jnp.maximum(_matmul_t(h, W1_ref[...]) + b1_ref[...], 0.0)
    mean = jnp.mean(h, axis=0, keepdims=True)
    var = jnp.mean((h - mean) ** 2, axis=0, keepdims=True)
    h = (h - mean) * (g_ref[...] * lax.rsqrt(var + 1e-5)) + be_ref[...]
    h = _matmul_t(h, W2_ref[...]) + b2_ref[...]
    o_ref[...] = _matmul_t(h, Wfc_ref[...]) + bfc_ref[...]


def kernel(x, edge_index, W1, b1, gamma, beta, W2, b2, Wfc, bfc):
    x = x.astype(jnp.float32)
    ei = edge_index.astype(jnp.int32)
    src = ei[0].reshape(NW, NBLK, IBLK, CHUNK)
    dst = ei[1].reshape(NW, NBLK, IBLK, CHUNK)
    zero = jnp.zeros((N_NODES, NFEAT), jnp.float32)

    agg = _sc_aggregate(x, src, dst, zero)

    out = pl.pallas_call(
        _tc_mlp,
        out_shape=jax.ShapeDtypeStruct((N_NODES, NCLASS), jnp.float32),
    )(agg, W1, b1.reshape(1, -1), gamma.reshape(1, -1),
      beta.reshape(1, -1), W2, b2.reshape(1, -1), Wfc, bfc.reshape(1, -1))
    return out
